# Initial kernel scaffold; baseline (speedup 1.0000x reference)
import jax
import jax.numpy as jnp
from jax.experimental import pallas as pl


def kernel(ent_embs, rel_embs, w_neighbor, w_loop, gru_w_ih, gru_w_hh, gru_b_ih, gru_b_hh, conv_w, conv_b, fc_w, fc_b, src_all, dst_all, etype_all, triplets):
    raise NotImplementedError("write your pallas kernel here")


# speedup vs baseline: 1.0437x; 1.0437x over previous
import functools

import jax
import jax.numpy as jnp
from jax.experimental import pallas as pl
from jax.experimental.pallas import tpu as pltpu

_RRELU_SLOPE = (1.0 / 8.0 + 1.0 / 3.0) / 2.0
_BN_EVAL_SCALE = 1.0 / (1.0 + 1e-5) ** 0.5

_F32 = jnp.float32
_BF16 = jnp.bfloat16


def _pre_kernel(ent_ref, rel_ref, wn_ref, wl0_ref,
                entw0_ref, loop0_ref, relw0_ref, relw1_ref):
    wn0 = wn_ref[0]
    wn1 = wn_ref[1]
    ent = ent_ref[...]
    rel = rel_ref[...]
    entw0_ref[...] = jnp.dot(ent, wn0, preferred_element_type=_F32).astype(_BF16)
    loop0_ref[...] = jnp.dot(ent, wl0_ref[...], preferred_element_type=_F32)
    relw0_ref[...] = jnp.dot(rel, wn0, preferred_element_type=_F32).astype(_BF16)
    relw1_ref[...] = jnp.dot(rel, wn1, preferred_element_type=_F32).astype(_BF16)


def _rgcn_kernel(src_ref, typ_ref, dst_ref, entw0_ref, loop0_ref,
                 relw0_ref, relw1_ref, wl1_ref, wn1_ref, h2_ref):
    E = src_ref.shape[1]
    Np, Dp = entw0_ref.shape
    Rp = relw0_ref.shape[0]

    src_col = src_ref[0]
    typ_col = typ_ref[0]
    dst_row = dst_ref[0]

    src_oh = (src_col == jax.lax.broadcasted_iota(jnp.int32, (E, Np), 1)).astype(_BF16)
    typ_oh = (typ_col == jax.lax.broadcasted_iota(jnp.int32, (E, Rp), 1)).astype(_BF16)
    dst_oh = (jax.lax.broadcasted_iota(jnp.int32, (Np, E), 0) == dst_row).astype(_BF16)
    deg = jnp.sum(dst_oh.astype(_F32), axis=1, keepdims=True)
    inv_deg = 1.0 / jnp.maximum(deg, 1.0)

    msg0 = (jnp.dot(src_oh, entw0_ref[...], preferred_element_type=_F32)
            + jnp.dot(typ_oh, relw0_ref[...], preferred_element_type=_F32))
    agg0 = jnp.dot(dst_oh, msg0.astype(_BF16), preferred_element_type=_F32) * inv_deg
    node0 = loop0_ref[...] + agg0
    h1 = jnp.where(node0 >= 0.0, node0, node0 * _RRELU_SLOPE)
    h1b = h1.astype(_BF16)

    loop1 = jnp.dot(h1b, wl1_ref[...], preferred_element_type=_F32)
    src_h1 = jnp.dot(src_oh, h1b, preferred_element_type=_F32)
    msg1 = (jnp.dot(src_h1.astype(_BF16), wn1_ref[...], preferred_element_type=_F32)
            + jnp.dot(typ_oh, relw1_ref[...], preferred_element_type=_F32))
    agg1 = jnp.dot(dst_oh, msg1.astype(_BF16), preferred_element_type=_F32) * inv_deg
    node1 = loop1 + agg1
    h2 = jnp.where(node1 >= 0.0, node1, node1 * _RRELU_SLOPE)
    h2_ref[0] = h2.astype(_BF16)


def _gru_kernel(h2_ref, wih_ref, whh_ref, bih_ref, bhh_ref, e1_ref, state_ref, *, T):
    t = pl.program_id(1)
    Dp = wih_ref.shape[0]

    @pl.when(t == 0)
    def _():
        state_ref[...] = jnp.zeros_like(state_ref)

    x = h2_ref[0]
    gi = jnp.dot(x, wih_ref[...], preferred_element_type=_F32) + bih_ref[...]
    hprev = state_ref[...]
    gh = (jnp.dot(hprev.astype(_BF16), whh_ref[...], preferred_element_type=_F32)
          + bhh_ref[...])
    r = jax.nn.sigmoid(gi[:, 0:Dp] + gh[:, 0:Dp])
    z = jax.nn.sigmoid(gi[:, Dp:2 * Dp] + gh[:, Dp:2 * Dp])
    n = jnp.tanh(gi[:, 2 * Dp:3 * Dp] + r * gh[:, 2 * Dp:3 * Dp])
    state = (1.0 - z) * n + z * hprev
    state_ref[...] = state

    @pl.when(t == T - 1)
    def _():
        e1_ref[...] = jnp.tanh(state).astype(_BF16)


def _dec_kernel(sub_ref, relidx_ref, obj_ref, e1_ref, relb_ref, fcw_ref,
                fcb_ref, convwb_ref, scores_ref, loss_ref, *, C):
    BB = sub_ref.shape[0]
    Np, Dp = e1_ref.shape
    Rp = relb_ref.shape[0]

    lane_n = jax.lax.broadcasted_iota(jnp.int32, (BB, Np), 1)
    lane_r = jax.lax.broadcasted_iota(jnp.int32, (BB, Rp), 1)
    sub_oh = (sub_ref[...] == lane_n).astype(_BF16)
    rel_oh = (relidx_ref[...] == lane_r).astype(_BF16)
    obj_oh = (obj_ref[...] == lane_n).astype(_F32)

    e1 = jnp.dot(sub_oh, e1_ref[...], preferred_element_type=_F32) * _BN_EVAL_SCALE
    rl = jnp.dot(rel_oh, relb_ref[...], preferred_element_type=_F32) * _BN_EVAL_SCALE

    lane_d = jax.lax.broadcasted_iota(jnp.int32, (BB, Dp), 1)
    zero = jnp.zeros((BB, Dp), _F32)

    def prev_tap(x):
        return jnp.where(lane_d >= 1, pltpu.roll(x, 1, 1), zero)

    def next_tap(x):
        return jnp.where(lane_d < Dp - 1, pltpu.roll(x, Dp - 1, 1), zero)

    taps = (prev_tap(e1), e1, next_tap(e1), prev_tap(rl), rl, next_tap(rl))

    conv_parts = []
    for c in range(C):
        acc = convwb_ref[c, 0] * taps[0]
        for j in range(1, 6):
            acc = acc + convwb_ref[c, j] * taps[j]
        acc = acc + convwb_ref[c, 6]
        conv_parts.append(jnp.maximum(acc * _BN_EVAL_SCALE, 0.0).astype(_BF16))
    conv_flat = jnp.concatenate(conv_parts, axis=1)

    fc = jnp.dot(conv_flat, fcw_ref[...], preferred_element_type=_F32) + fcb_ref[...]
    query = jnp.maximum(fc * _BN_EVAL_SCALE, 0.0).astype(_BF16)

    scores = jax.lax.dot_general(query, e1_ref[...], (((1,), (1,)), ((), ())),
                                 preferred_element_type=_F32)
    scores_ref[...] = scores

    m = jnp.max(scores, axis=-1, keepdims=True)
    lse = m + jnp.log(jnp.sum(jnp.exp(scores - m), axis=-1, keepdims=True))
    tgt = jnp.sum(scores * obj_oh, axis=-1, keepdims=True)
    loss_ref[0] = jnp.sum(lse - tgt, axis=0, keepdims=True)


def kernel(ent_embs, rel_embs, w_neighbor, w_loop, gru_w_ih, gru_w_hh,
           gru_b_ih, gru_b_hh, conv_w, conv_b, fc_w, fc_b,
           src_all, dst_all, etype_all, triplets):
    N, D = ent_embs.shape
    R = rel_embs.shape[0]
    T, E = src_all.shape
    B = triplets.shape[0]
    C = conv_w.shape[0]

    entb = ent_embs.astype(_BF16)
    relb = rel_embs.astype(_BF16)
    wnb = w_neighbor.astype(_BF16)
    wl0b = w_loop[0].astype(_BF16)
    wl1b = w_loop[1].astype(_BF16)
    wn1b = w_neighbor[1].astype(_BF16)

    entw0, loop0, relw0, relw1 = pl.pallas_call(
        _pre_kernel,
        out_shape=(jax.ShapeDtypeStruct((N, D), _BF16),
                   jax.ShapeDtypeStruct((N, D), _F32),
                   jax.ShapeDtypeStruct((R, D), _BF16),
                   jax.ShapeDtypeStruct((R, D), _BF16)),
    )(entb, relb, wnb, wl0b)

    src_arr = src_all.astype(jnp.int32)[:, :, None]
    typ_arr = etype_all.astype(jnp.int32)[:, :, None]
    dst_arr = dst_all.astype(jnp.int32)[:, None, :]

    h2_all = pl.pallas_call(
        _rgcn_kernel,
        out_shape=jax.ShapeDtypeStruct((T, N, D), _BF16),
        grid=(T,),
        in_specs=[
            pl.BlockSpec((1, E, 1), lambda t: (t, 0, 0)),
            pl.BlockSpec((1, E, 1), lambda t: (t, 0, 0)),
            pl.BlockSpec((1, 1, E), lambda t: (t, 0, 0)),
            pl.BlockSpec((N, D), lambda t: (0, 0)),
            pl.BlockSpec((N, D), lambda t: (0, 0)),
            pl.BlockSpec((R, D), lambda t: (0, 0)),
            pl.BlockSpec((R, D), lambda t: (0, 0)),
            pl.BlockSpec((D, D), lambda t: (0, 0)),
            pl.BlockSpec((D, D), lambda t: (0, 0)),
        ],
        out_specs=pl.BlockSpec((1, N, D), lambda t: (t, 0, 0)),
        compiler_params=pltpu.CompilerParams(dimension_semantics=("parallel",)),
    )(src_arr, typ_arr, dst_arr, entw0, loop0, relw0, relw1, wl1b, wn1b)

    w3i = gru_w_ih.reshape(3, D, D)
    w3h = gru_w_hh.reshape(3, D, D)
    wih = jnp.transpose(w3i, (2, 0, 1)).reshape(D, 3 * D).astype(_BF16)
    whh = jnp.transpose(w3h, (2, 0, 1)).reshape(D, 3 * D).astype(_BF16)
    bih = gru_b_ih.reshape(1, 3 * D)
    bhh = gru_b_hh.reshape(1, 3 * D)

    NB = 2
    BN = N // NB
    e1_all = pl.pallas_call(
        functools.partial(_gru_kernel, T=T),
        out_shape=jax.ShapeDtypeStruct((N, D), _BF16),
        grid=(NB, T),
        in_specs=[
            pl.BlockSpec((1, BN, D), lambda n, t: (t, n, 0)),
            pl.BlockSpec((D, 3 * D), lambda n, t: (0, 0)),
            pl.BlockSpec((D, 3 * D), lambda n, t: (0, 0)),
            pl.BlockSpec((1, 3 * D), lambda n, t: (0, 0)),
            pl.BlockSpec((1, 3 * D), lambda n, t: (0, 0)),
        ],
        out_specs=pl.BlockSpec((BN, D), lambda n, t: (n, 0)),
        scratch_shapes=[pltpu.VMEM((BN, D), _F32)],
        compiler_params=pltpu.CompilerParams(
            dimension_semantics=("parallel", "arbitrary")),
    )(h2_all, wih, whh, bih, bhh)

    sub = triplets[:, 0].astype(jnp.int32).reshape(B, 1)
    rel_idx = triplets[:, 1].astype(jnp.int32).reshape(B, 1)
    obj = triplets[:, 2].astype(jnp.int32).reshape(B, 1)

    fcw = fc_w.reshape(C * D, D).astype(_BF16)
    fcb = fc_b.reshape(1, D)
    conv_wb = jnp.concatenate([conv_w.reshape(C, 6), conv_b[:, None]], axis=1)

    NQ = 2
    BB = B // NQ
    scores, loss_parts = pl.pallas_call(
        functools.partial(_dec_kernel, C=C),
        out_shape=(jax.ShapeDtypeStruct((B, N), _F32),
                   jax.ShapeDtypeStruct((NQ, 1, 1), _F32)),
        grid=(NQ,),
        in_specs=[
            pl.BlockSpec((BB, 1), lambda b: (b, 0)),
            pl.BlockSpec((BB, 1), lambda b: (b, 0)),
            pl.BlockSpec((BB, 1), lambda b: (b, 0)),
            pl.BlockSpec((N, D), lambda b: (0, 0)),
            pl.BlockSpec((R, D), lambda b: (0, 0)),
            pl.BlockSpec((C * D, D), lambda b: (0, 0)),
            pl.BlockSpec((1, D), lambda b: (0, 0)),
            pl.BlockSpec(memory_space=pltpu.MemorySpace.SMEM),
        ],
        out_specs=(pl.BlockSpec((BB, N), lambda b: (b, 0)),
                   pl.BlockSpec((1, 1, 1), lambda b: (b, 0, 0))),
        compiler_params=pltpu.CompilerParams(dimension_semantics=("parallel",)),
    )(sub, rel_idx, obj, e1_all, relb, fcw, fcb, conv_wb)

    loss = jnp.sum(loss_parts) / B
    return loss, scores
